# baseline (speedup 1.0000x reference)
import functools
import math

import jax
import jax.numpy as jnp
from jax import lax
from jax.experimental import pallas as pl
from jax.experimental.pallas import tpu as pltpu
from jax.experimental.pallas import tpu_sc as plsc

B_, S_, D_ = 1, 2048, 1024
H_, HKV_, HD_ = 16, 4, 64
E_, K_, DFF_ = 8, 2, 2816
CAP_ = 640
THETA_ = 10000.0
T_ = B_ * S_
G_ = HKV_
REP_ = H_ // HKV_
HALF_ = HD_ // 2
Z_ = E_ * CAP_
EROWS_ = (E_ + 1) * CAP_

BS1_ = 256
BQ_ = 512
BK_ = 512
DFB_ = 1408

NC_, NS_ = 2, 16
NW_ = NC_ * NS_
LOG_THETA_ = math.log(THETA_)


def _rope_full(x, pos_f, nh):
    n = nh * HD_
    j = lax.broadcasted_iota(jnp.int32, (1, n), 1)
    within = j % HD_
    sel = within < HALF_
    fidx = jnp.where(sel, within, within - HALF_).astype(jnp.float32)
    fvec = jnp.exp(fidx * (-LOG_THETA_ / HALF_))
    ang = pos_f * fvec
    c = jnp.cos(ang)
    s = jnp.sin(ang)
    partner = jnp.where(sel, pltpu.roll(x, n - HALF_, 1), pltpu.roll(x, HALF_, 1))
    return jnp.where(sel, x * c - partner * s, x * c + partner * s)


def _k1_body(hs_ref, wn_ref, pos_ref, wq_ref, wk_ref, wv_ref,
             q_ref, k_ref, v_ref):
    x = hs_ref[...]
    xn = x * lax.rsqrt(jnp.mean(x * x, axis=1, keepdims=True) + 1e-6)
    xn = xn * wn_ref[...]
    pos_f = pos_ref[...].astype(jnp.float32)
    q = jnp.dot(xn, wq_ref[...], preferred_element_type=jnp.float32)
    k = jnp.dot(xn, wk_ref[...], preferred_element_type=jnp.float32)
    v = jnp.dot(xn, wv_ref[...], preferred_element_type=jnp.float32)
    q_ref[...] = _rope_full(q, pos_f, H_)
    k_ref[...] = _rope_full(k, pos_f, HKV_)
    v_ref[...] = v


def _k2_body(q_ref, k_ref, v_ref, o_ref):
    i = pl.program_id(1)
    scale = 1.0 / math.sqrt(HD_)
    qhs = [q_ref[0, :, jh * HD_:(jh + 1) * HD_] * scale for jh in range(REP_)]

    def step(kb, carry, masked):
        kblk = k_ref[0, pl.ds(kb * BK_, BK_), :]
        vblk = v_ref[0, pl.ds(kb * BK_, BK_), :]
        out = []
        for jh in range(REP_):
            m, l, acc = carry[jh]
            s = lax.dot_general(qhs[jh], kblk, (((1,), (1,)), ((), ())),
                                preferred_element_type=jnp.float32)
            if masked:
                qpos = lax.broadcasted_iota(jnp.int32, (BQ_, BK_), 0)
                kpos = lax.broadcasted_iota(jnp.int32, (BQ_, BK_), 1)
                s = jnp.where(qpos >= kpos, s, -1e9)
            mnew = jnp.maximum(m, jnp.max(s, axis=1, keepdims=True))
            p = jnp.exp(s - mnew)
            alpha = jnp.exp(m - mnew)
            l2 = l * alpha + jnp.sum(p, axis=1, keepdims=True)
            acc2 = acc * alpha + jnp.dot(p, vblk,
                                         preferred_element_type=jnp.float32)
            out.append((mnew, l2, acc2))
        return tuple(out)

    init = tuple((jnp.full((BQ_, 1), -1e30, jnp.float32),
                  jnp.zeros((BQ_, 1), jnp.float32),
                  jnp.zeros((BQ_, HD_), jnp.float32)) for _ in range(REP_))
    carry = lax.fori_loop(0, i, lambda kb, c: step(kb, c, False), init)
    carry = step(i, carry, True)
    o_ref[0] = jnp.concatenate([acc / l for (m, l, acc) in carry], axis=1)


def _k3_body(hs_ref, ao_ref, wo_ref, wn_ref, wg_ref,
             hs2_ref, xn2_ref, oh0_ref, oh1_ref, w0_ref, w1_ref):
    hs2 = hs_ref[...] + jnp.dot(ao_ref[...], wo_ref[...],
                                preferred_element_type=jnp.float32)
    hs2_ref[...] = hs2
    xn2 = hs2 * lax.rsqrt(jnp.mean(hs2 * hs2, axis=1, keepdims=True) + 1e-6)
    xn2 = xn2 * wn_ref[...]
    xn2_ref[...] = xn2
    logits = jnp.dot(xn2, wg_ref[...], preferred_element_type=jnp.float32)
    iot = lax.broadcasted_iota(jnp.int32, (BS1_, E_), 1)
    m1 = jnp.max(logits, axis=1, keepdims=True)
    i1 = jnp.min(jnp.where(logits == m1, iot, E_), axis=1, keepdims=True)
    masked = jnp.where(iot == i1, -1e30, logits)
    m2 = jnp.max(masked, axis=1, keepdims=True)
    i2 = jnp.min(jnp.where(masked == m2, iot, E_), axis=1, keepdims=True)
    ew = jnp.exp(m2 - m1)
    w0_ref[...] = 1.0 / (1.0 + ew)
    w1_ref[...] = ew / (1.0 + ew)
    oh0_ref[...] = (iot == i1).astype(jnp.float32)
    oh1_ref[...] = (iot == i2).astype(jnp.float32)


def _k4_body(oh0_ref, oh1_ref, w0_ref, w1_ref,
             scat0_ref, scat1_ref, vsrc0_ref, vsrc1_ref,
             vw0_ref, vw1_ref, comb0_ref, comb1_ref):
    oh0 = oh0_ref[...]
    oh1 = oh1_ref[...]
    both = oh0 + oh1
    ck = 512
    tot = jnp.zeros((1, E_), jnp.float32)
    chunks = []
    for c in range(T_ // ck):
        bc = both[c * ck:(c + 1) * ck, :]
        r = lax.broadcasted_iota(jnp.int32, (ck, ck), 0)
        cc = lax.broadcasted_iota(jnp.int32, (ck, ck), 1)
        ltri = (cc < r).astype(jnp.float32)
        part = jnp.dot(ltri, bc, preferred_element_type=jnp.float32)
        chunks.append(part + tot)
        tot = tot + jnp.sum(bc, axis=0, keepdims=True)
    A = jnp.concatenate(chunks, axis=0)
    iot = lax.broadcasted_iota(jnp.int32, (T_, E_), 1).astype(jnp.float32)
    tvec = lax.broadcasted_iota(jnp.int32, (T_, 1), 0)
    for oh, scat_ref, vsrc_ref, vw_ref, comb_ref, w_ref in (
            (oh0, scat0_ref, vsrc0_ref, vw0_ref, comb0_ref, w0_ref),
            (oh1, scat1_ref, vsrc1_ref, vw1_ref, comb1_ref, w1_ref)):
        pos = jnp.sum(A * oh, axis=1, keepdims=True)
        ei = jnp.sum(iot * oh, axis=1, keepdims=True).astype(jnp.int32)
        keep = pos < float(CAP_)
        posc = jnp.where(keep, pos, 0.0).astype(jnp.int32)
        scat = ei * CAP_ + posc
        scat_ref[...] = scat
        vsrc_ref[...] = jnp.where(keep, tvec + 1, 0)
        vw_ref[...] = jnp.where(keep, w_ref[...], 0.0)
        comb_ref[...] = jnp.where(keep, scat, Z_)


def _k5_body(disp_ref, w1_ref, w3_ref, w2_ref, wrow_ref, out_ref):
    e = pl.program_id(0)
    kb = pl.program_id(1)

    @pl.when(e < E_)
    def _():
        x = disp_ref[...].astype(jnp.bfloat16)
        h1 = jnp.dot(x, w1_ref[0].astype(jnp.bfloat16),
                     preferred_element_type=jnp.float32)
        h3 = jnp.dot(x, w3_ref[0].astype(jnp.bfloat16),
                     preferred_element_type=jnp.float32)
        tt = (h1 * jax.nn.sigmoid(h1)) * h3
        part = jnp.dot(tt.astype(jnp.bfloat16), w2_ref[0].astype(jnp.bfloat16),
                       preferred_element_type=jnp.float32)

        @pl.when(kb == 0)
        def _():
            out_ref[...] = part

        @pl.when(kb == DFF_ // DFB_ - 1)
        def _():
            out_ref[...] = (out_ref[...] + part) * wrow_ref[0]

        @pl.when(jnp.logical_and(kb > 0, kb < DFF_ // DFB_ - 1))
        def _():
            out_ref[...] = out_ref[...] + part

    @pl.when(e == E_)
    def _():
        out_ref[...] = jnp.zeros_like(out_ref)


def _sca_body(scat0, scat1, vsrc0, vsrc1, vw0, vw1,
              src_out, wrow_out,
              s_sc0, s_sc1, s_v0, s_v1, s_w0, s_w1, a_src, a_w):
    cid = lax.axis_index("c")
    sid = lax.axis_index("s")

    @pl.when(jnp.logical_and(cid == 0, sid == 0))
    def _():
        pltpu.sync_copy(scat0, s_sc0)
        pltpu.sync_copy(scat1, s_sc1)
        pltpu.sync_copy(vsrc0, s_v0)
        pltpu.sync_copy(vsrc1, s_v1)
        pltpu.sync_copy(vw0, s_w0)
        pltpu.sync_copy(vw1, s_w1)

        def zero(i, _):
            a_src[pl.ds(i * 16, 16)] = jnp.zeros((16,), jnp.int32)
            a_w[pl.ds(i * 16, 16)] = jnp.zeros((16,), jnp.float32)
            return 0

        lax.fori_loop(0, Z_ // 16, zero, 0)

        def scat(i, _):
            i0 = s_sc0[pl.ds(i * 16, 16)]
            i1 = s_sc1[pl.ds(i * 16, 16)]
            plsc.addupdate_scatter(a_src, [i0], s_v0[pl.ds(i * 16, 16)])
            plsc.addupdate_scatter(a_src, [i1], s_v1[pl.ds(i * 16, 16)])
            plsc.addupdate_scatter(a_w, [i0], s_w0[pl.ds(i * 16, 16)])
            plsc.addupdate_scatter(a_w, [i1], s_w1[pl.ds(i * 16, 16)])
            return 0

        lax.fori_loop(0, T_ // 16, scat, 0)
        pltpu.sync_copy(a_src, src_out)
        pltpu.sync_copy(a_w, wrow_out.at[pl.ds(0, Z_)])
        pltpu.sync_copy(a_w.at[pl.ds(0, CAP_)], wrow_out.at[pl.ds(Z_, CAP_)])


_SCB_NCH = 10
_SCB_CH = Z_ // NW_ // _SCB_NCH


def _scb_body(src_hbm, xn2_hbm, disp_hbm, idx_v, rows0_v, rows1_v,
              gs0, gs1, ws0, ws1):
    wid = lax.axis_index("s") * NC_ + lax.axis_index("c")
    rows_per = Z_ // NW_
    base = wid * rows_per
    for c in range(_SCB_NCH):
        pltpu.sync_copy(src_hbm.at[pl.ds(base + c * _SCB_CH, _SCB_CH)],
                        idx_v.at[c])
        for t in range(_SCB_CH // 16):
            v = idx_v[c, pl.ds(t * 16, 16)]
            idx_v[c, pl.ds(t * 16, 16)] = jnp.maximum(v - 1, 0)
    bufs = (rows0_v, rows1_v)
    gsems = (gs0, gs1)
    wsems = (ws0, ws1)
    writes = [None, None]
    for c in range(_SCB_NCH):
        b = c % 2
        if writes[b] is not None:
            writes[b].wait()
        pltpu.async_copy(xn2_hbm.at[idx_v.at[c]], bufs[b], gsems[b]).wait()
        writes[b] = pltpu.async_copy(
            bufs[b], disp_hbm.at[pl.ds(base + c * _SCB_CH, _SCB_CH)], wsems[b])
    writes[0].wait()
    writes[1].wait()


def _scc_body(eout_hbm, comb0_hbm, comb1_hbm, moe0_hbm, moe1_hbm,
              idx_v, buf_v, sem):
    wid = lax.axis_index("s") * NC_ + lax.axis_index("c")
    tok_per = T_ // NW_
    base = wid * tok_per
    for comb_hbm, moe_hbm in ((comb0_hbm, moe0_hbm), (comb1_hbm, moe1_hbm)):
        pltpu.sync_copy(comb_hbm.at[pl.ds(base, tok_per)], idx_v)
        pltpu.async_copy(eout_hbm.at[idx_v], buf_v, sem).wait()
        pltpu.sync_copy(buf_v, moe_hbm.at[pl.ds(base, tok_per)])


def _k6_body(hs2_ref, m0_ref, m1_ref, out_ref):
    out_ref[...] = hs2_ref[...] + m0_ref[...] + m1_ref[...]


@functools.lru_cache(maxsize=1)
def _sc_kernels():
    mesh = plsc.VectorSubcoreMesh(core_axis_name="c", subcore_axis_name="s",
                                  num_cores=NC_, num_subcores=NS_)
    sca = pl.kernel(
        _sca_body,
        out_type=(jax.ShapeDtypeStruct((Z_,), jnp.int32),
                  jax.ShapeDtypeStruct((EROWS_,), jnp.float32)),
        mesh=mesh,
        compiler_params=pltpu.CompilerParams(needs_layout_passes=False),
        scratch_types=(pltpu.VMEM((T_,), jnp.int32), pltpu.VMEM((T_,), jnp.int32),
                       pltpu.VMEM((T_,), jnp.int32), pltpu.VMEM((T_,), jnp.int32),
                       pltpu.VMEM((T_,), jnp.float32), pltpu.VMEM((T_,), jnp.float32),
                       pltpu.VMEM((Z_,), jnp.int32), pltpu.VMEM((Z_,), jnp.float32)),
    )
    scb = pl.kernel(
        _scb_body,
        out_type=jax.ShapeDtypeStruct((Z_, D_), jnp.float32),
        mesh=mesh,
        scratch_types=(pltpu.VMEM((_SCB_NCH, _SCB_CH), jnp.int32),
                       pltpu.VMEM((_SCB_CH, D_), jnp.float32),
                       pltpu.VMEM((_SCB_CH, D_), jnp.float32),
                       pltpu.SemaphoreType.DMA, pltpu.SemaphoreType.DMA,
                       pltpu.SemaphoreType.DMA, pltpu.SemaphoreType.DMA),
    )
    scc = pl.kernel(
        _scc_body,
        out_type=(jax.ShapeDtypeStruct((T_, D_), jnp.float32),
                  jax.ShapeDtypeStruct((T_, D_), jnp.float32)),
        mesh=mesh,
        scratch_types=(pltpu.VMEM((T_ // NW_,), jnp.int32),
                       pltpu.VMEM((T_ // NW_, D_), jnp.float32),
                       pltpu.SemaphoreType.DMA),
    )
    return sca, scb, scc


def kernel(hidden_states, w_seq_norm, w_ffn_norm, Wq, Wk, Wv, Wo, Wg,
           W1, W3, W2, position_ids):
    f32 = jnp.float32
    hs = hidden_states.reshape(T_, D_)
    pos2d = position_ids.reshape(T_, 1)
    wn1 = w_seq_norm.reshape(1, D_)
    wn2 = w_ffn_norm.reshape(1, D_)

    nb = T_ // BS1_
    q, k, v = pl.pallas_call(
        _k1_body,
        grid=(nb,),
        in_specs=[
            pl.BlockSpec((BS1_, D_), lambda i: (i, 0)),
            pl.BlockSpec((1, D_), lambda i: (0, 0)),
            pl.BlockSpec((BS1_, 1), lambda i: (i, 0)),
            pl.BlockSpec((D_, H_ * HD_), lambda i: (0, 0)),
            pl.BlockSpec((D_, HKV_ * HD_), lambda i: (0, 0)),
            pl.BlockSpec((D_, HKV_ * HD_), lambda i: (0, 0)),
        ],
        out_specs=[
            pl.BlockSpec((BS1_, H_ * HD_), lambda i: (i, 0)),
            pl.BlockSpec((BS1_, HKV_ * HD_), lambda i: (i, 0)),
            pl.BlockSpec((BS1_, HKV_ * HD_), lambda i: (i, 0)),
        ],
        out_shape=[jax.ShapeDtypeStruct((T_, H_ * HD_), f32),
                   jax.ShapeDtypeStruct((T_, HKV_ * HD_), f32),
                   jax.ShapeDtypeStruct((T_, HKV_ * HD_), f32)],
    )(hs, wn1, pos2d, Wq, Wk, Wv)

    q4 = q.reshape(T_, G_, REP_ * HD_).transpose(1, 0, 2)
    kT = k.reshape(T_, G_, HD_).transpose(1, 0, 2)
    vT = v.reshape(T_, G_, HD_).transpose(1, 0, 2)

    ao4 = pl.pallas_call(
        _k2_body,
        grid=(G_, T_ // BQ_),
        in_specs=[
            pl.BlockSpec((1, BQ_, REP_ * HD_), lambda g, i: (g, i, 0)),
            pl.BlockSpec((1, T_, HD_), lambda g, i: (g, 0, 0)),
            pl.BlockSpec((1, T_, HD_), lambda g, i: (g, 0, 0)),
        ],
        out_specs=pl.BlockSpec((1, BQ_, REP_ * HD_), lambda g, i: (g, i, 0)),
        out_shape=jax.ShapeDtypeStruct((G_, T_, REP_ * HD_), f32),
    )(q4, kT, vT)

    ao = ao4.transpose(1, 0, 2).reshape(T_, H_ * HD_)

    hs2, xn2, oh0, oh1, w0, w1 = pl.pallas_call(
        _k3_body,
        grid=(nb,),
        in_specs=[
            pl.BlockSpec((BS1_, D_), lambda i: (i, 0)),
            pl.BlockSpec((BS1_, H_ * HD_), lambda i: (i, 0)),
            pl.BlockSpec((H_ * HD_, D_), lambda i: (0, 0)),
            pl.BlockSpec((1, D_), lambda i: (0, 0)),
            pl.BlockSpec((D_, E_), lambda i: (0, 0)),
        ],
        out_specs=[
            pl.BlockSpec((BS1_, D_), lambda i: (i, 0)),
            pl.BlockSpec((BS1_, D_), lambda i: (i, 0)),
            pl.BlockSpec((BS1_, E_), lambda i: (i, 0)),
            pl.BlockSpec((BS1_, E_), lambda i: (i, 0)),
            pl.BlockSpec((BS1_, 1), lambda i: (i, 0)),
            pl.BlockSpec((BS1_, 1), lambda i: (i, 0)),
        ],
        out_shape=[jax.ShapeDtypeStruct((T_, D_), f32),
                   jax.ShapeDtypeStruct((T_, D_), f32),
                   jax.ShapeDtypeStruct((T_, E_), f32),
                   jax.ShapeDtypeStruct((T_, E_), f32),
                   jax.ShapeDtypeStruct((T_, 1), f32),
                   jax.ShapeDtypeStruct((T_, 1), f32)],
    )(hs, ao, Wo, wn2, Wg)

    i32 = jnp.int32
    meta_out = pl.pallas_call(
        _k4_body,
        grid=(1,),
        in_specs=[pl.BlockSpec((T_, E_), lambda i: (0, 0)),
                  pl.BlockSpec((T_, E_), lambda i: (0, 0)),
                  pl.BlockSpec((T_, 1), lambda i: (0, 0)),
                  pl.BlockSpec((T_, 1), lambda i: (0, 0))],
        out_specs=[pl.BlockSpec((T_, 1), lambda i: (0, 0))] * 8,
        out_shape=[jax.ShapeDtypeStruct((T_, 1), i32),
                   jax.ShapeDtypeStruct((T_, 1), i32),
                   jax.ShapeDtypeStruct((T_, 1), i32),
                   jax.ShapeDtypeStruct((T_, 1), i32),
                   jax.ShapeDtypeStruct((T_, 1), f32),
                   jax.ShapeDtypeStruct((T_, 1), f32),
                   jax.ShapeDtypeStruct((T_, 1), i32),
                   jax.ShapeDtypeStruct((T_, 1), i32)],
    )(oh0, oh1, w0, w1)
    scat0, scat1, vsrc0, vsrc1, vw0, vw1, comb0, comb1 = (
        a.reshape(T_) for a in meta_out)

    sca, scb, scc = _sc_kernels()
    src, wrow = sca(scat0, scat1, vsrc0, vsrc1, vw0, vw1)
    disp = scb(src, xn2)

    eout = pl.pallas_call(
        _k5_body,
        grid=(E_ + 1, DFF_ // DFB_),
        in_specs=[
            pl.BlockSpec((CAP_, D_), lambda e, kb: (jnp.minimum(e, E_ - 1), 0)),
            pl.BlockSpec((1, D_, DFB_),
                         lambda e, kb: (jnp.minimum(e, E_ - 1), 0, kb)),
            pl.BlockSpec((1, D_, DFB_),
                         lambda e, kb: (jnp.minimum(e, E_ - 1), 0, kb)),
            pl.BlockSpec((1, DFB_, D_),
                         lambda e, kb: (jnp.minimum(e, E_ - 1), kb, 0)),
            pl.BlockSpec((1, CAP_, 1), lambda e, kb: (e, 0, 0)),
        ],
        out_specs=pl.BlockSpec((CAP_, D_), lambda e, kb: (e, 0)),
        out_shape=jax.ShapeDtypeStruct((EROWS_, D_), f32),
    )(disp, W1, W3, W2, wrow.reshape(E_ + 1, CAP_, 1))

    moe0, moe1 = scc(eout, comb0, comb1)
    out = pl.pallas_call(
        _k6_body,
        grid=(nb,),
        in_specs=[pl.BlockSpec((BS1_, D_), lambda i: (i, 0))] * 3,
        out_specs=pl.BlockSpec((BS1_, D_), lambda i: (i, 0)),
        out_shape=jax.ShapeDtypeStruct((T_, D_), f32),
    )(hs2, moe0, moe1)
    return out.reshape(B_, S_, D_)


# speedup vs baseline: 1.0164x; 1.0164x over previous
import functools
import math

import jax
import jax.numpy as jnp
from jax import lax
from jax.experimental import pallas as pl
from jax.experimental.pallas import tpu as pltpu
from jax.experimental.pallas import tpu_sc as plsc

B_, S_, D_ = 1, 2048, 1024
H_, HKV_, HD_ = 16, 4, 64
E_, K_, DFF_ = 8, 2, 2816
CAP_ = 640
THETA_ = 10000.0
T_ = B_ * S_
G_ = HKV_
REP_ = H_ // HKV_
HALF_ = HD_ // 2
Z_ = E_ * CAP_
EROWS_ = (E_ + 1) * CAP_

BS1_ = 256
BQ_ = 512
BK_ = 512
DFB_ = 1408

NC_, NS_ = 2, 16
NW_ = NC_ * NS_
LOG_THETA_ = math.log(THETA_)


def _rope_full(x, pos_f, nh):
    n = nh * HD_
    j = lax.broadcasted_iota(jnp.int32, (1, n), 1)
    within = j % HD_
    sel = within < HALF_
    fidx = jnp.where(sel, within, within - HALF_).astype(jnp.float32)
    fvec = jnp.exp(fidx * (-LOG_THETA_ / HALF_))
    ang = pos_f * fvec
    c = jnp.cos(ang)
    s = jnp.sin(ang)
    partner = jnp.where(sel, pltpu.roll(x, n - HALF_, 1), pltpu.roll(x, HALF_, 1))
    return jnp.where(sel, x * c - partner * s, x * c + partner * s)


def _k1_body(hs_ref, wn_ref, pos_ref, wq_ref, wk_ref, wv_ref,
             q_ref, k_ref, v_ref):
    x = hs_ref[...]
    xn = x * lax.rsqrt(jnp.mean(x * x, axis=1, keepdims=True) + 1e-6)
    xn = xn * wn_ref[...]
    pos_f = pos_ref[...].astype(jnp.float32)
    q = jnp.dot(xn, wq_ref[...], preferred_element_type=jnp.float32)
    k = jnp.dot(xn, wk_ref[...], preferred_element_type=jnp.float32)
    v = jnp.dot(xn, wv_ref[...], preferred_element_type=jnp.float32)
    q_ref[...] = _rope_full(q, pos_f, H_)
    k_ref[...] = _rope_full(k, pos_f, HKV_)
    v_ref[...] = v


def _k2_body(q_ref, k_ref, v_ref, o_ref):
    i = pl.program_id(1)
    scale = 1.0 / math.sqrt(HD_)
    qhs = [q_ref[0, :, jh * HD_:(jh + 1) * HD_] * scale for jh in range(REP_)]

    def step(kb, carry, masked):
        kblk = k_ref[0, pl.ds(kb * BK_, BK_), :]
        vblk = v_ref[0, pl.ds(kb * BK_, BK_), :]
        out = []
        for jh in range(REP_):
            m, l, acc = carry[jh]
            s = lax.dot_general(qhs[jh], kblk, (((1,), (1,)), ((), ())),
                                preferred_element_type=jnp.float32)
            if masked:
                qpos = lax.broadcasted_iota(jnp.int32, (BQ_, BK_), 0)
                kpos = lax.broadcasted_iota(jnp.int32, (BQ_, BK_), 1)
                s = jnp.where(qpos >= kpos, s, -1e9)
            mnew = jnp.maximum(m, jnp.max(s, axis=1, keepdims=True))
            p = jnp.exp(s - mnew)
            alpha = jnp.exp(m - mnew)
            l2 = l * alpha + jnp.sum(p, axis=1, keepdims=True)
            acc2 = acc * alpha + jnp.dot(p, vblk,
                                         preferred_element_type=jnp.float32)
            out.append((mnew, l2, acc2))
        return tuple(out)

    init = tuple((jnp.full((BQ_, 1), -1e30, jnp.float32),
                  jnp.zeros((BQ_, 1), jnp.float32),
                  jnp.zeros((BQ_, HD_), jnp.float32)) for _ in range(REP_))
    carry = lax.fori_loop(0, i, lambda kb, c: step(kb, c, False), init)
    carry = step(i, carry, True)
    o_ref[0] = jnp.concatenate([acc / l for (m, l, acc) in carry], axis=1)


def _k3_body(hs_ref, ao_ref, wo_ref, wn_ref, wg_ref,
             hs2_ref, xn2_ref, oh0_ref, oh1_ref, w0_ref, w1_ref):
    hs2 = hs_ref[...] + jnp.dot(ao_ref[...], wo_ref[...],
                                preferred_element_type=jnp.float32)
    hs2_ref[...] = hs2
    xn2 = hs2 * lax.rsqrt(jnp.mean(hs2 * hs2, axis=1, keepdims=True) + 1e-6)
    xn2 = xn2 * wn_ref[...]
    xn2_ref[...] = xn2
    logits = jnp.dot(xn2, wg_ref[...], preferred_element_type=jnp.float32)
    iot = lax.broadcasted_iota(jnp.int32, (BS1_, E_), 1)
    m1 = jnp.max(logits, axis=1, keepdims=True)
    i1 = jnp.min(jnp.where(logits == m1, iot, E_), axis=1, keepdims=True)
    masked = jnp.where(iot == i1, -1e30, logits)
    m2 = jnp.max(masked, axis=1, keepdims=True)
    i2 = jnp.min(jnp.where(masked == m2, iot, E_), axis=1, keepdims=True)
    ew = jnp.exp(m2 - m1)
    w0_ref[...] = 1.0 / (1.0 + ew)
    w1_ref[...] = ew / (1.0 + ew)
    oh0_ref[...] = (iot == i1).astype(jnp.float32)
    oh1_ref[...] = (iot == i2).astype(jnp.float32)


def _k4_body(oh0_ref, oh1_ref, w0_ref, w1_ref,
             scat0_ref, scat1_ref, vsrc0_ref, vsrc1_ref,
             vw0_ref, vw1_ref, comb0_ref, comb1_ref):
    oh0 = oh0_ref[...]
    oh1 = oh1_ref[...]
    both = oh0 + oh1
    ck = 512
    tot = jnp.zeros((1, E_), jnp.float32)
    chunks = []
    for c in range(T_ // ck):
        bc = both[c * ck:(c + 1) * ck, :]
        r = lax.broadcasted_iota(jnp.int32, (ck, ck), 0)
        cc = lax.broadcasted_iota(jnp.int32, (ck, ck), 1)
        ltri = (cc < r).astype(jnp.float32)
        part = jnp.dot(ltri, bc, preferred_element_type=jnp.float32)
        chunks.append(part + tot)
        tot = tot + jnp.sum(bc, axis=0, keepdims=True)
    A = jnp.concatenate(chunks, axis=0)
    iot = lax.broadcasted_iota(jnp.int32, (T_, E_), 1).astype(jnp.float32)
    tvec = lax.broadcasted_iota(jnp.int32, (T_, 1), 0)
    for oh, scat_ref, vsrc_ref, vw_ref, comb_ref, w_ref in (
            (oh0, scat0_ref, vsrc0_ref, vw0_ref, comb0_ref, w0_ref),
            (oh1, scat1_ref, vsrc1_ref, vw1_ref, comb1_ref, w1_ref)):
        pos = jnp.sum(A * oh, axis=1, keepdims=True)
        ei = jnp.sum(iot * oh, axis=1, keepdims=True).astype(jnp.int32)
        keep = pos < float(CAP_)
        posc = jnp.where(keep, pos, 0.0).astype(jnp.int32)
        scat = ei * CAP_ + posc
        scat_ref[...] = scat
        vsrc_ref[...] = jnp.where(keep, tvec + 1, 0)
        vw_ref[...] = jnp.where(keep, w_ref[...], 0.0)
        comb_ref[...] = jnp.where(keep, scat, Z_)


def _k5_body(disp_ref, w1_ref, w3_ref, w2_ref, wrow_ref, out_ref):
    e = pl.program_id(0)
    kb = pl.program_id(1)

    @pl.when(e < E_)
    def _():
        x = disp_ref[...].astype(jnp.bfloat16)
        h1 = jnp.dot(x, w1_ref[0].astype(jnp.bfloat16),
                     preferred_element_type=jnp.float32)
        h3 = jnp.dot(x, w3_ref[0].astype(jnp.bfloat16),
                     preferred_element_type=jnp.float32)
        tt = (h1 * jax.nn.sigmoid(h1)) * h3
        part = jnp.dot(tt.astype(jnp.bfloat16), w2_ref[0].astype(jnp.bfloat16),
                       preferred_element_type=jnp.float32)

        @pl.when(kb == 0)
        def _():
            out_ref[...] = part

        @pl.when(kb == DFF_ // DFB_ - 1)
        def _():
            out_ref[...] = (out_ref[...] + part) * wrow_ref[0]

        @pl.when(jnp.logical_and(kb > 0, kb < DFF_ // DFB_ - 1))
        def _():
            out_ref[...] = out_ref[...] + part

    @pl.when(e == E_)
    def _():
        out_ref[...] = jnp.zeros_like(out_ref)


def _sca_body(scat0, scat1, vsrc0, vsrc1, vw0, vw1,
              src_out, wrow_out,
              s_sc0, s_sc1, s_v0, s_v1, s_w0, s_w1, a_src, a_w):
    cid = lax.axis_index("c")
    sid = lax.axis_index("s")

    @pl.when(jnp.logical_and(cid == 0, sid == 0))
    def _():
        pltpu.sync_copy(scat0, s_sc0)
        pltpu.sync_copy(scat1, s_sc1)
        pltpu.sync_copy(vsrc0, s_v0)
        pltpu.sync_copy(vsrc1, s_v1)
        pltpu.sync_copy(vw0, s_w0)
        pltpu.sync_copy(vw1, s_w1)

        def zero(i, _):
            a_src[pl.ds(i * 16, 16)] = jnp.zeros((16,), jnp.int32)
            a_w[pl.ds(i * 16, 16)] = jnp.zeros((16,), jnp.float32)
            return 0

        lax.fori_loop(0, Z_ // 16, zero, 0)

        def scat(i, _):
            i0 = s_sc0[pl.ds(i * 16, 16)]
            i1 = s_sc1[pl.ds(i * 16, 16)]
            plsc.addupdate_scatter(a_src, [i0], s_v0[pl.ds(i * 16, 16)])
            plsc.addupdate_scatter(a_src, [i1], s_v1[pl.ds(i * 16, 16)])
            plsc.addupdate_scatter(a_w, [i0], s_w0[pl.ds(i * 16, 16)])
            plsc.addupdate_scatter(a_w, [i1], s_w1[pl.ds(i * 16, 16)])
            return 0

        lax.fori_loop(0, T_ // 16, scat, 0)
        pltpu.sync_copy(a_src, src_out)
        pltpu.sync_copy(a_w, wrow_out.at[pl.ds(0, Z_)])
        pltpu.sync_copy(a_w.at[pl.ds(0, CAP_)], wrow_out.at[pl.ds(Z_, CAP_)])


_SCB_NCH = 4
_SCB_CH = Z_ // NW_ // _SCB_NCH


def _scb_body(src_hbm, xn2_hbm, disp_hbm, idx_v, rows0_v, rows1_v,
              gs0, gs1, ws0, ws1):
    wid = lax.axis_index("s") * NC_ + lax.axis_index("c")
    rows_per = Z_ // NW_
    base = wid * rows_per
    for c in range(_SCB_NCH):
        pltpu.sync_copy(src_hbm.at[pl.ds(base + c * _SCB_CH, _SCB_CH)],
                        idx_v.at[c])
        for t in range(_SCB_CH // 16):
            v = idx_v[c, pl.ds(t * 16, 16)]
            idx_v[c, pl.ds(t * 16, 16)] = jnp.maximum(v - 1, 0)
    bufs = (rows0_v, rows1_v)
    gsems = (gs0, gs1)
    wsems = (ws0, ws1)
    writes = [None, None]
    for c in range(_SCB_NCH):
        b = c % 2
        if writes[b] is not None:
            writes[b].wait()
        pltpu.async_copy(xn2_hbm.at[idx_v.at[c]], bufs[b], gsems[b]).wait()
        writes[b] = pltpu.async_copy(
            bufs[b], disp_hbm.at[pl.ds(base + c * _SCB_CH, _SCB_CH)], wsems[b])
    writes[0].wait()
    writes[1].wait()


def _scc_body(eout_hbm, comb0_hbm, comb1_hbm, moe0_hbm, moe1_hbm,
              idx_v, buf_v, sem):
    wid = lax.axis_index("s") * NC_ + lax.axis_index("c")
    tok_per = T_ // NW_
    base = wid * tok_per
    for comb_hbm, moe_hbm in ((comb0_hbm, moe0_hbm), (comb1_hbm, moe1_hbm)):
        pltpu.sync_copy(comb_hbm.at[pl.ds(base, tok_per)], idx_v)
        pltpu.async_copy(eout_hbm.at[idx_v], buf_v, sem).wait()
        pltpu.sync_copy(buf_v, moe_hbm.at[pl.ds(base, tok_per)])


def _k6_body(hs2_ref, m0_ref, m1_ref, out_ref):
    out_ref[...] = hs2_ref[...] + m0_ref[...] + m1_ref[...]


@functools.lru_cache(maxsize=1)
def _sc_kernels():
    mesh = plsc.VectorSubcoreMesh(core_axis_name="c", subcore_axis_name="s",
                                  num_cores=NC_, num_subcores=NS_)
    sca = pl.kernel(
        _sca_body,
        out_type=(jax.ShapeDtypeStruct((Z_,), jnp.int32),
                  jax.ShapeDtypeStruct((EROWS_,), jnp.float32)),
        mesh=mesh,
        compiler_params=pltpu.CompilerParams(needs_layout_passes=False),
        scratch_types=(pltpu.VMEM((T_,), jnp.int32), pltpu.VMEM((T_,), jnp.int32),
                       pltpu.VMEM((T_,), jnp.int32), pltpu.VMEM((T_,), jnp.int32),
                       pltpu.VMEM((T_,), jnp.float32), pltpu.VMEM((T_,), jnp.float32),
                       pltpu.VMEM((Z_,), jnp.int32), pltpu.VMEM((Z_,), jnp.float32)),
    )
    scb = pl.kernel(
        _scb_body,
        out_type=jax.ShapeDtypeStruct((Z_, D_), jnp.float32),
        mesh=mesh,
        scratch_types=(pltpu.VMEM((_SCB_NCH, _SCB_CH), jnp.int32),
                       pltpu.VMEM((_SCB_CH, D_), jnp.float32),
                       pltpu.VMEM((_SCB_CH, D_), jnp.float32),
                       pltpu.SemaphoreType.DMA, pltpu.SemaphoreType.DMA,
                       pltpu.SemaphoreType.DMA, pltpu.SemaphoreType.DMA),
    )
    scc = pl.kernel(
        _scc_body,
        out_type=(jax.ShapeDtypeStruct((T_, D_), jnp.float32),
                  jax.ShapeDtypeStruct((T_, D_), jnp.float32)),
        mesh=mesh,
        scratch_types=(pltpu.VMEM((T_ // NW_,), jnp.int32),
                       pltpu.VMEM((T_ // NW_, D_), jnp.float32),
                       pltpu.SemaphoreType.DMA),
    )
    return sca, scb, scc


def kernel(hidden_states, w_seq_norm, w_ffn_norm, Wq, Wk, Wv, Wo, Wg,
           W1, W3, W2, position_ids):
    f32 = jnp.float32
    hs = hidden_states.reshape(T_, D_)
    pos2d = position_ids.reshape(T_, 1)
    wn1 = w_seq_norm.reshape(1, D_)
    wn2 = w_ffn_norm.reshape(1, D_)

    nb = T_ // BS1_
    q, k, v = pl.pallas_call(
        _k1_body,
        grid=(nb,),
        in_specs=[
            pl.BlockSpec((BS1_, D_), lambda i: (i, 0)),
            pl.BlockSpec((1, D_), lambda i: (0, 0)),
            pl.BlockSpec((BS1_, 1), lambda i: (i, 0)),
            pl.BlockSpec((D_, H_ * HD_), lambda i: (0, 0)),
            pl.BlockSpec((D_, HKV_ * HD_), lambda i: (0, 0)),
            pl.BlockSpec((D_, HKV_ * HD_), lambda i: (0, 0)),
        ],
        out_specs=[
            pl.BlockSpec((BS1_, H_ * HD_), lambda i: (i, 0)),
            pl.BlockSpec((BS1_, HKV_ * HD_), lambda i: (i, 0)),
            pl.BlockSpec((BS1_, HKV_ * HD_), lambda i: (i, 0)),
        ],
        out_shape=[jax.ShapeDtypeStruct((T_, H_ * HD_), f32),
                   jax.ShapeDtypeStruct((T_, HKV_ * HD_), f32),
                   jax.ShapeDtypeStruct((T_, HKV_ * HD_), f32)],
    )(hs, wn1, pos2d, Wq, Wk, Wv)

    q4 = q.reshape(T_, G_, REP_ * HD_).transpose(1, 0, 2)
    kT = k.reshape(T_, G_, HD_).transpose(1, 0, 2)
    vT = v.reshape(T_, G_, HD_).transpose(1, 0, 2)

    ao4 = pl.pallas_call(
        _k2_body,
        grid=(G_, T_ // BQ_),
        in_specs=[
            pl.BlockSpec((1, BQ_, REP_ * HD_), lambda g, i: (g, i, 0)),
            pl.BlockSpec((1, T_, HD_), lambda g, i: (g, 0, 0)),
            pl.BlockSpec((1, T_, HD_), lambda g, i: (g, 0, 0)),
        ],
        out_specs=pl.BlockSpec((1, BQ_, REP_ * HD_), lambda g, i: (g, i, 0)),
        out_shape=jax.ShapeDtypeStruct((G_, T_, REP_ * HD_), f32),
    )(q4, kT, vT)

    ao = ao4.transpose(1, 0, 2).reshape(T_, H_ * HD_)

    hs2, xn2, oh0, oh1, w0, w1 = pl.pallas_call(
        _k3_body,
        grid=(nb,),
        in_specs=[
            pl.BlockSpec((BS1_, D_), lambda i: (i, 0)),
            pl.BlockSpec((BS1_, H_ * HD_), lambda i: (i, 0)),
            pl.BlockSpec((H_ * HD_, D_), lambda i: (0, 0)),
            pl.BlockSpec((1, D_), lambda i: (0, 0)),
            pl.BlockSpec((D_, E_), lambda i: (0, 0)),
        ],
        out_specs=[
            pl.BlockSpec((BS1_, D_), lambda i: (i, 0)),
            pl.BlockSpec((BS1_, D_), lambda i: (i, 0)),
            pl.BlockSpec((BS1_, E_), lambda i: (i, 0)),
            pl.BlockSpec((BS1_, E_), lambda i: (i, 0)),
            pl.BlockSpec((BS1_, 1), lambda i: (i, 0)),
            pl.BlockSpec((BS1_, 1), lambda i: (i, 0)),
        ],
        out_shape=[jax.ShapeDtypeStruct((T_, D_), f32),
                   jax.ShapeDtypeStruct((T_, D_), f32),
                   jax.ShapeDtypeStruct((T_, E_), f32),
                   jax.ShapeDtypeStruct((T_, E_), f32),
                   jax.ShapeDtypeStruct((T_, 1), f32),
                   jax.ShapeDtypeStruct((T_, 1), f32)],
    )(hs, ao, Wo, wn2, Wg)

    i32 = jnp.int32
    meta_out = pl.pallas_call(
        _k4_body,
        grid=(1,),
        in_specs=[pl.BlockSpec((T_, E_), lambda i: (0, 0)),
                  pl.BlockSpec((T_, E_), lambda i: (0, 0)),
                  pl.BlockSpec((T_, 1), lambda i: (0, 0)),
                  pl.BlockSpec((T_, 1), lambda i: (0, 0))],
        out_specs=[pl.BlockSpec((T_, 1), lambda i: (0, 0))] * 8,
        out_shape=[jax.ShapeDtypeStruct((T_, 1), i32),
                   jax.ShapeDtypeStruct((T_, 1), i32),
                   jax.ShapeDtypeStruct((T_, 1), i32),
                   jax.ShapeDtypeStruct((T_, 1), i32),
                   jax.ShapeDtypeStruct((T_, 1), f32),
                   jax.ShapeDtypeStruct((T_, 1), f32),
                   jax.ShapeDtypeStruct((T_, 1), i32),
                   jax.ShapeDtypeStruct((T_, 1), i32)],
    )(oh0, oh1, w0, w1)
    scat0, scat1, vsrc0, vsrc1, vw0, vw1, comb0, comb1 = (
        a.reshape(T_) for a in meta_out)

    sca, scb, scc = _sc_kernels()
    src, wrow = sca(scat0, scat1, vsrc0, vsrc1, vw0, vw1)
    disp = scb(src, xn2)

    eout = pl.pallas_call(
        _k5_body,
        grid=(E_ + 1, DFF_ // DFB_),
        in_specs=[
            pl.BlockSpec((CAP_, D_), lambda e, kb: (jnp.minimum(e, E_ - 1), 0)),
            pl.BlockSpec((1, D_, DFB_),
                         lambda e, kb: (jnp.minimum(e, E_ - 1), 0, kb)),
            pl.BlockSpec((1, D_, DFB_),
                         lambda e, kb: (jnp.minimum(e, E_ - 1), 0, kb)),
            pl.BlockSpec((1, DFB_, D_),
                         lambda e, kb: (jnp.minimum(e, E_ - 1), kb, 0)),
            pl.BlockSpec((1, CAP_, 1), lambda e, kb: (e, 0, 0)),
        ],
        out_specs=pl.BlockSpec((CAP_, D_), lambda e, kb: (e, 0)),
        out_shape=jax.ShapeDtypeStruct((EROWS_, D_), f32),
    )(disp, W1, W3, W2, wrow.reshape(E_ + 1, CAP_, 1))

    moe0, moe1 = scc(eout, comb0, comb1)
    out = pl.pallas_call(
        _k6_body,
        grid=(nb,),
        in_specs=[pl.BlockSpec((BS1_, D_), lambda i: (i, 0))] * 3,
        out_specs=pl.BlockSpec((BS1_, D_), lambda i: (i, 0)),
        out_shape=jax.ShapeDtypeStruct((T_, D_), f32),
    )(hs2, moe0, moe1)
    return out.reshape(B_, S_, D_)
